# baseline (speedup 1.0000x reference)
import functools

import jax
import jax.numpy as jnp
from jax import lax
from jax.experimental import pallas as pl
from jax.experimental.pallas import tpu as pltpu
from jax.experimental.pallas import tpu_sc as plsc

_BETA = 0.5


def _argmin_kernel(bm, bk, n_k, n_m, big_idx, loss_scale,
                   x_sq_ref, w_sq_ref, x_ref, w2_ref,
                   tok_ref, loss_ref, acc_ref):
    m = pl.program_id(0)
    x_sq = x_sq_ref[...]
    best_d = None
    best_i = None
    for c in range(n_k):
        dotneg = lax.dot_general(
            x_ref[...], w2_ref[pl.ds(c * bk, bk), :],
            dimension_numbers=(((1,), (1,)), ((), ())),
            preferred_element_type=jnp.float32,
        )
        sq = x_sq[:, None] + w_sq_ref[pl.ds(c * bk, bk)][None, :] + dotneg
        dist = jnp.sqrt(jnp.maximum(sq, 0.0))
        tmin = jnp.min(dist, axis=1)
        iota = lax.broadcasted_iota(jnp.int32, (bm, bk), 1)
        targ = jnp.min(jnp.where(dist == tmin[:, None], iota, big_idx),
                       axis=1) + c * bk
        if c == 0:
            best_d, best_i = tmin, targ
        else:
            better = tmin < best_d
            best_d = jnp.where(better, tmin, best_d)
            best_i = jnp.where(better, targ, best_i)

    tok_ref[0, 0, :] = best_i
    part = jnp.sum(best_d * best_d)

    @pl.when(m == 0)
    def _():
        acc_ref[0, 0] = part

    @pl.when(m > 0)
    def _():
        acc_ref[0, 0] = acc_ref[0, 0] + part

    @pl.when(m == n_m - 1)
    def _():
        loss_ref[0, 0] = acc_ref[0, 0] * loss_scale


def _tc_argmin(flat_x, w2, x_sq, w_sq, bm=512, bk=2048):
    m_total, d = flat_x.shape
    k_total = w2.shape[0]
    n_m = m_total // bm
    n_k = k_total // bk
    loss_scale = (1.0 + _BETA) / (m_total * d)
    tokens3, loss = pl.pallas_call(
        functools.partial(_argmin_kernel, bm, bk, n_k, n_m, k_total,
                          loss_scale),
        grid=(n_m,),
        in_specs=[
            pl.BlockSpec((bm,), lambda m: (m,)),
            pl.BlockSpec((k_total,), lambda m: (0,)),
            pl.BlockSpec((bm, d), lambda m: (m, 0)),
            pl.BlockSpec((k_total, d), lambda m: (0, 0)),
        ],
        out_specs=[
            pl.BlockSpec((1, 1, bm), lambda m: (m, 0, 0)),
            pl.BlockSpec(memory_space=pltpu.SMEM),
        ],
        out_shape=[
            jax.ShapeDtypeStruct((n_m, 1, bm), jnp.int32),
            jax.ShapeDtypeStruct((1, 1), jnp.float32),
        ],
        scratch_shapes=[
            pltpu.SMEM((1, 1), jnp.float32),
        ],
        compiler_params=pltpu.CompilerParams(
            vmem_limit_bytes=100 * 1024 * 1024,
        ),
    )(x_sq, w_sq, flat_x, w2)
    return tokens3.reshape(m_total), loss[0, 0]


def _sc_gather(table, idx):
    v, d = table.shape
    m_total = idx.shape[0]
    nw = 32
    b_per_w = m_total // nw
    chunk = 192
    n_chunks = b_per_w // chunk
    mesh = plsc.VectorSubcoreMesh(core_axis_name="c", subcore_axis_name="s")

    @functools.partial(
        pl.kernel,
        mesh=mesh,
        out_type=jax.ShapeDtypeStruct((m_total, d), jnp.float32),
        scratch_types=[
            pltpu.VMEM((chunk,), jnp.int32),
            pltpu.VMEM((chunk, d), jnp.float32),
            pltpu.SemaphoreType.DMA,
        ],
    )
    def gather_kernel(table_hbm, idx_hbm, out_hbm, idx_v, rows_v, sem):
        wid = lax.axis_index("s") * 2 + lax.axis_index("c")
        base = wid * b_per_w
        for j in range(n_chunks):
            off = base + j * chunk
            pltpu.sync_copy(idx_hbm.at[pl.ds(off, chunk)], idx_v)
            pltpu.async_copy(table_hbm.at[idx_v], rows_v, sem).wait()
            pltpu.sync_copy(rows_v, out_hbm.at[pl.ds(off, chunk)])

    return gather_kernel(table, idx)


def kernel(x, w):
    b, n, d = x.shape
    flat_x = x.reshape(-1, d)
    x_sq = jnp.sum(flat_x * flat_x, axis=1)
    w_sq = jnp.sum(w * w, axis=1)
    w2 = -2.0 * w
    tokens_flat, vq_loss = _tc_argmin(flat_x, w2, x_sq, w_sq)
    quantized = _sc_gather(w, tokens_flat).reshape(b, n, d)
    tokens = tokens_flat.reshape(b, n)
    quantized_st = x + (quantized - x)
    return (tokens, quantized_st, vq_loss)


# speedup vs baseline: 1.3420x; 1.3420x over previous
import functools

import jax
import jax.numpy as jnp
from jax import lax
from jax.experimental import pallas as pl
from jax.experimental.pallas import tpu as pltpu
from jax.experimental.pallas import tpu_sc as plsc

_BETA = 0.5


def _argmin_kernel(bm, bk, n_k, n_m, big_idx, loss_scale,
                   x_sq_ref, w_sq_ref, x_ref, w2_ref,
                   tok_ref, loss_ref, sq_ref, acc_ref):
    m = pl.program_id(0)
    x_sq = x_sq_ref[...]
    run_min = None
    for c in range(n_k):
        dotneg = lax.dot_general(
            x_ref[...], w2_ref[pl.ds(c * bk, bk), :],
            dimension_numbers=(((1,), (1,)), ((), ())),
            preferred_element_type=jnp.float32,
        )
        sq = x_sq[:, None] + w_sq_ref[pl.ds(c * bk, bk)][None, :] + dotneg
        sq_ref[:, pl.ds(c * bk, bk)] = sq
        cmin = jnp.min(sq, axis=1)
        run_min = cmin if c == 0 else jnp.minimum(run_min, cmin)
    best_d = jnp.sqrt(jnp.maximum(run_min, 0.0))
    v0i = lax.bitcast_convert_type(best_d * best_d, jnp.int32)
    hi = None
    for koff in range(-4, 5):
        vk = lax.bitcast_convert_type(jnp.maximum(v0i + koff, 0),
                                      jnp.float32)
        ok = jnp.sqrt(jnp.maximum(vk, 0.0)) == best_d
        cand = jnp.where(ok, vk, -jnp.inf)
        hi = cand if hi is None else jnp.maximum(hi, cand)
    best_i = None
    for c in range(n_k):
        sq = sq_ref[:, pl.ds(c * bk, bk)]
        iota = lax.broadcasted_iota(jnp.int32, (bm, bk), 1)
        targ = jnp.min(jnp.where(sq <= hi[:, None], iota, big_idx),
                       axis=1) + c * bk
        best_i = targ if c == 0 else jnp.minimum(best_i, targ)

    tok_ref[0, 0, :] = best_i
    part = jnp.sum(best_d * best_d)

    @pl.when(m == 0)
    def _():
        acc_ref[0, 0] = part

    @pl.when(m > 0)
    def _():
        acc_ref[0, 0] = acc_ref[0, 0] + part

    @pl.when(m == n_m - 1)
    def _():
        loss_ref[0, 0] = acc_ref[0, 0] * loss_scale


def _tc_argmin(flat_x, w2, x_sq, w_sq, bm=512, bk=2048):
    m_total, d = flat_x.shape
    k_total = w2.shape[0]
    n_m = m_total // bm
    n_k = k_total // bk
    loss_scale = (1.0 + _BETA) / (m_total * d)
    tokens3, loss = pl.pallas_call(
        functools.partial(_argmin_kernel, bm, bk, n_k, n_m, k_total,
                          loss_scale),
        grid=(n_m,),
        in_specs=[
            pl.BlockSpec((bm,), lambda m: (m,)),
            pl.BlockSpec((k_total,), lambda m: (0,)),
            pl.BlockSpec((bm, d), lambda m: (m, 0)),
            pl.BlockSpec((k_total, d), lambda m: (0, 0)),
        ],
        out_specs=[
            pl.BlockSpec((1, 1, bm), lambda m: (m, 0, 0)),
            pl.BlockSpec(memory_space=pltpu.SMEM),
        ],
        out_shape=[
            jax.ShapeDtypeStruct((n_m, 1, bm), jnp.int32),
            jax.ShapeDtypeStruct((1, 1), jnp.float32),
        ],
        scratch_shapes=[
            pltpu.VMEM((bm, k_total), jnp.float32),
            pltpu.SMEM((1, 1), jnp.float32),
        ],
        compiler_params=pltpu.CompilerParams(
            vmem_limit_bytes=100 * 1024 * 1024,
        ),
    )(x_sq, w_sq, flat_x, w2)
    return tokens3.reshape(m_total), loss[0, 0]


def _sc_gather(table, idx):
    v, d = table.shape
    m_total = idx.shape[0]
    nw = 32
    b_per_w = m_total // nw
    chunk = 192
    n_chunks = b_per_w // chunk
    mesh = plsc.VectorSubcoreMesh(core_axis_name="c", subcore_axis_name="s")

    @functools.partial(
        pl.kernel,
        mesh=mesh,
        out_type=jax.ShapeDtypeStruct((m_total, d), jnp.float32),
        scratch_types=[
            pltpu.VMEM((chunk,), jnp.int32),
            pltpu.VMEM((chunk, d), jnp.float32),
            pltpu.SemaphoreType.DMA,
        ],
    )
    def gather_kernel(table_hbm, idx_hbm, out_hbm, idx_v, rows_v, sem):
        wid = lax.axis_index("s") * 2 + lax.axis_index("c")
        base = wid * b_per_w
        for j in range(n_chunks):
            off = base + j * chunk
            pltpu.sync_copy(idx_hbm.at[pl.ds(off, chunk)], idx_v)
            pltpu.async_copy(table_hbm.at[idx_v], rows_v, sem).wait()
            pltpu.sync_copy(rows_v, out_hbm.at[pl.ds(off, chunk)])

    return gather_kernel(table, idx)


def kernel(x, w):
    b, n, d = x.shape
    flat_x = x.reshape(-1, d)
    x_sq = jnp.sum(flat_x * flat_x, axis=1)
    w_sq = jnp.sum(w * w, axis=1)
    w2 = -2.0 * w
    tokens_flat, vq_loss = _tc_argmin(flat_x, w2, x_sq, w_sq)
    quantized = _sc_gather(w, tokens_flat).reshape(b, n, d)
    tokens = tokens_flat.reshape(b, n)
    quantized_st = x + (quantized - x)
    return (tokens, quantized_st, vq_loss)
